# Initial kernel scaffold; baseline (speedup 1.0000x reference)
import jax
import jax.numpy as jnp
from jax.experimental import pallas as pl


def kernel(pred_box_0, pred_cnf_0, pred_cls_0, pred_box_1, pred_cnf_1, pred_cls_1, pred_box_2, pred_cnf_2, pred_cls_2, targets):
    raise NotImplementedError("write your pallas kernel here")


# speedup vs baseline: 1.1344x; 1.1344x over previous
import functools
import math

import jax
import jax.numpy as jnp
import numpy as np
from jax import lax
from jax.experimental import pallas as pl
from jax.experimental.pallas import tpu as pltpu
from jax.experimental.pallas import tpu_sc as plsc

f32 = jnp.float32
i32 = jnp.int32

CLASS_NUM = 80
BATCH = 16
STRIDES = (8.0, 16.0, 32.0)
GRID_SIZES = (80, 40, 20)
ANCHORS = (
    ((12.0, 16.0), (19.0, 36.0), (40.0, 28.0)),
    ((36.0, 75.0), (76.0, 55.0), (72.0, 146.0)),
    ((142.0, 110.0), (192.0, 243.0), (459.0, 401.0)),
)
ANCH_TOPK = 4.0
BOX_WT, OBJ_WT, CLS_WT = 0.05, 0.7, 0.3
OBJ_SCALE = (4.0, 1.0, 0.4)
EPS = 1e-07
OFF_X = (0.0, 0.5, 0.0, -0.5, 0.0)
OFF_Y = (0.0, 0.0, 0.5, 0.0, -0.5)

NT = 640
NCAND = 5 * 3 * NT
CAND = 10240
ROWS128 = CAND // 128
NW = 32
PER_TILE = CAND // NW
CHUNK = 80
NCHUNK = PER_TILE // CHUNK
NCELLS = tuple(BATCH * 3 * g * g for g in GRID_SIZES)
GRID_STEPS = 10
CBLK = ROWS128 // GRID_STEPS
GPAD_ROWS = (240, 64, 32)
GPAD_TOT = tuple(r * GRID_STEPS for r in GPAD_ROWS)


def _sel_const(idx, vals):
    out = jnp.full(idx.shape, vals[-1], dtype=f32)
    for k in range(len(vals) - 2, -1, -1):
        out = jnp.where(idx == k, jnp.float32(vals[k]), out)
    return out


def _assign(fields, lvl, c):
    cls_c, x1, y1, x2, y2 = fields
    s = STRIDES[lvl]
    G = GRID_SIZES[lvl]
    o = c // 1920
    a = (c // 640) % 3
    n = c % 640
    b = n // 40
    valid = cls_c > -1.0
    whx = x2 - x1
    why = y2 - y1
    gx = (x1 + whx * 0.5) * jnp.float32(1.0 / s)
    gy = (y1 + why * 0.5) * jnp.float32(1.0 / s)
    gwx = whx * jnp.float32(1.0 / s)
    gwy = why * jnp.float32(1.0 / s)
    aw = _sel_const(a, tuple(ANCHORS[lvl][k][0] / s for k in range(3)))
    ah = _sel_const(a, tuple(ANCHORS[lvl][k][1] / s for k in range(3)))
    rx = gwx / aw
    ry = gwy / ah
    afilt = (jnp.maximum(jnp.maximum(rx, 1.0 / rx),
                         jnp.maximum(ry, 1.0 / ry)) < ANCH_TOPK)
    fgx = gx - jnp.floor(gx)
    fgy = gy - jnp.floor(gy)
    gxi = jnp.float32(G) - gx
    gyi = jnp.float32(G) - gy
    fgxi = gxi - jnp.floor(gxi)
    fgyi = gyi - jnp.floor(gyi)
    one = jnp.float32(1.0)
    zero = jnp.float32(0.0)
    ga = jnp.where((fgx < 0.5) & (gx > 1.0), one, zero)
    gb = jnp.where((fgy < 0.5) & (gy > 1.0), one, zero)
    gia = jnp.where((fgxi < 0.5) & (gxi > 1.0), one, zero)
    gib = jnp.where((fgyi < 0.5) & (gyi > 1.0), one, zero)
    jm = jnp.where(o == 0, one,
         jnp.where(o == 1, ga,
         jnp.where(o == 2, gb,
         jnp.where(o == 3, gia, gib))))
    afilt_f = jnp.where(afilt, one, zero)
    valid_f = jnp.where(valid, one, zero)
    pad_f = jnp.where(c < NCAND, one, zero)
    mask = jm * afilt_f * valid_f * pad_f
    ox = _sel_const(o, OFF_X)
    oy = _sel_const(o, OFF_Y)
    gi = jnp.clip((gx - ox).astype(i32), 0, G - 1)
    gj = jnp.clip((gy - oy).astype(i32), 0, G - 1)
    row = ((b * 3 + a) * G + gj) * G + gi
    tbx = gx - gi.astype(f32)
    tby = gy - gj.astype(f32)
    label = cls_c.astype(i32)
    return mask, row, (tbx, tby, gwx, gwy), (aw, ah), label


def _assign_kernel(cls_r, x1_r, y1_r, x2_r, y2_r,
                   r0, r1, r2, q0, q1, q2, w0, w1, w2):
    fields = (cls_r[...], x1_r[...], y1_r[...], x2_r[...], y2_r[...])
    c = (lax.broadcasted_iota(i32, (ROWS128, 128), 0) * 128
         + lax.broadcasted_iota(i32, (ROWS128, 128), 1))
    routs = (r0, r1, r2)
    qouts = (q0, q1, q2)
    wouts = (w0, w1, w2)
    for lvl in range(3):
        _, row, _, _, _ = _assign(fields, lvl, c)
        routs[lvl][...] = row
        qouts[lvl][...] = row // 4
        wouts[lvl][...] = row // 16


def _assign_call(fields):
    return pl.pallas_call(
        _assign_kernel,
        out_shape=tuple(jax.ShapeDtypeStruct((ROWS128, 128), i32)
                        for _ in range(9)),
    )(*fields)


def _sc_gather_body(r0, q0, w0, r1, q1, w1, r2, q2, w2,
                    tb0, tc0, tf0, tb1, tc1, tf1, tb2, tc2, tf2,
                    ob0, oc0, of0, ob1, oc1, of1, ob2, oc2, of2,
                    idx_v, qdx_v, wdx_v, box_v, cls_v, cnf_v, sem):
    wid = lax.axis_index("s") * 2 + lax.axis_index("c")
    base = wid * PER_TILE
    levels = ((r0, q0, w0, tb0, tc0, tf0, ob0, oc0, of0),
              (r1, q1, w1, tb1, tc1, tf1, ob1, oc1, of1),
              (r2, q2, w2, tb2, tc2, tf2, ob2, oc2, of2))
    for (r, q, w, tb, tc, tf, ob, oc, of) in levels:
        pltpu.sync_copy(r.at[pl.ds(wid * NCHUNK, NCHUNK)], idx_v)
        pltpu.sync_copy(q.at[pl.ds(wid * NCHUNK, NCHUNK)], qdx_v)
        pltpu.sync_copy(w.at[pl.ds(wid * NCHUNK, NCHUNK)], wdx_v)
        handles = []
        for k in range(NCHUNK):
            ki = jnp.int32(k)
            handles.append(pltpu.async_copy(
                tb.at[qdx_v.at[ki]], box_v.at[pl.ds(k * CHUNK, CHUNK)], sem))
            handles.append(pltpu.async_copy(
                tc.at[idx_v.at[ki]], cls_v.at[pl.ds(k * CHUNK, CHUNK)], sem))
            handles.append(pltpu.async_copy(
                tf.at[wdx_v.at[ki]], cnf_v.at[pl.ds(k * CHUNK, CHUNK)], sem))
        for h in handles:
            h.wait()
        pltpu.sync_copy(box_v, ob.at[pl.ds(base, PER_TILE)])
        pltpu.sync_copy(cls_v, oc.at[pl.ds(base, PER_TILE)])
        pltpu.sync_copy(cnf_v, of.at[pl.ds(base, PER_TILE)])


def _sc_gather_call(rows, rows4, rows16, box_fl, cls_fl, cnf_fl):
    mesh = plsc.VectorSubcoreMesh(core_axis_name="c", subcore_axis_name="s")
    out_type = []
    for lvl in range(3):
        out_type += [jax.ShapeDtypeStruct((CAND, 16), f32),
                     jax.ShapeDtypeStruct((CAND, CLASS_NUM), f32),
                     jax.ShapeDtypeStruct((CAND, 16), f32)]
    fn = pl.kernel(
        _sc_gather_body,
        out_type=tuple(out_type),
        mesh=mesh,
        compiler_params=pltpu.CompilerParams(use_tc_tiling_on_sc=False),
        scratch_types=[
            pltpu.VMEM((NCHUNK, CHUNK), i32),
            pltpu.VMEM((NCHUNK, CHUNK), i32),
            pltpu.VMEM((NCHUNK, CHUNK), i32),
            pltpu.VMEM((PER_TILE, 16), f32),
            pltpu.VMEM((PER_TILE, CLASS_NUM), f32),
            pltpu.VMEM((PER_TILE, 16), f32),
            pltpu.SemaphoreType.DMA,
        ],
    )
    return fn(rows[0], rows4[0], rows16[0],
              rows[1], rows4[1], rows16[1],
              rows[2], rows4[2], rows16[2],
              box_fl[0], cls_fl[0], cnf_fl[0],
              box_fl[1], cls_fl[1], cnf_fl[1],
              box_fl[2], cls_fl[2], cnf_fl[2])


_ATAN_C = (0.99997726, -0.33262347, 0.19354346,
           -0.11643287, 0.05265332, -0.01172120)


def _atan_pos(z):
    inv = z > 1.0
    t = jnp.where(inv, 1.0 / z, z)
    s = t * t
    p = jnp.float32(_ATAN_C[5])
    for ck in _ATAN_C[4::-1]:
        p = p * s + jnp.float32(ck)
    p = p * t
    return jnp.where(inv, jnp.float32(math.pi / 2) - p, p)


def _ciou(px, py, pw, ph, tx, ty, tw, th):
    x1a, x1b = px - pw * 0.5, px + pw * 0.5
    y1a, y1b = py - ph * 0.5, py + ph * 0.5
    x2a, x2b = tx - tw * 0.5, tx + tw * 0.5
    y2a, y2b = ty - th * 0.5, ty + th * 0.5
    xi = jnp.minimum(x1b, x2b) - jnp.maximum(x1a, x2a)
    yi = jnp.minimum(y1b, y2b) - jnp.maximum(y1a, y2a)
    inter = jnp.clip(xi, 0.0, None) * jnp.clip(yi, 0.0, None)
    a1 = (x1b - x1a) * (y1b - y1a)
    a2 = (x2b - x2a) * (y2b - y2a)
    union = a1 + a2 - inter
    iou = inter / (union + EPS)
    cxs = jnp.maximum(x1b, x2b) - jnp.minimum(x1a, x2a)
    cys = jnp.maximum(y1b, y2b) - jnp.minimum(y1a, y2a)
    diag = cxs * cxs + cys * cys + EPS
    cx1 = (x1b + x1a) * 0.5
    cy1 = (y1b + y1a) * 0.5
    cx2 = (x2b + x2a) * 0.5
    cy2 = (y2b + y2a) * 0.5
    cent = (cx1 - cx2) ** 2 + (cy1 - cy2) ** 2
    at = (_atan_pos((x1b - x1a) / (y1b - y1a + EPS))
          - _atan_pos((x2b - x2a) / (y2b - y2a + EPS)))
    v = jnp.float32(4.0 / math.pi ** 2) * at * at
    alpha = v / (v - iou + (1.0 + EPS))
    return iou - (cent / diag + v * alpha)


def _softplus_part(x):
    return jnp.clip(x, 0.0, None) + jnp.log(1.0 + jnp.exp(-jnp.abs(x)))


_Q_CNT, _Q_BOX, _Q_SPC, _Q_XLC, _Q_CORR, _Q_GSP = range(6)


def _loss_kernel(cls_r, x1_r, y1_r, x2_r, y2_r,
                 bx0, bx1, bx2, cf0, cf1, cf2, cl0, cl1, cl2,
                 g0, g1, g2, final_ref, acc):
    step = pl.program_id(0)

    @pl.when(step == 0)
    def _init():
        for k in range(18):
            acc[k] = jnp.float32(0.0)
        final_ref[...] = jnp.zeros((8, 128), f32)

    fields = (cls_r[...], x1_r[...], y1_r[...], x2_r[...], y2_r[...])
    c = ((lax.broadcasted_iota(i32, (CBLK, 128), 0) + step * CBLK) * 128
         + lax.broadcasted_iota(i32, (CBLK, 128), 1))
    boxg = (bx0, bx1, bx2)
    cnfg = (cf0, cf1, cf2)
    clsg = (cl0, cl1, cl2)
    grids = (g0, g1, g2)
    lane16 = lax.broadcasted_iota(i32, (CBLK, 128, 16), 2)
    for lvl in range(3):
        mf, row, (tbx, tby, gwx, gwy), (aw, ah), label = _assign(
            fields, lvl, c)
        pb16 = boxg[lvl][...]
        sub4 = (row % 4)[:, :, None]
        pbc = [jnp.sum(jnp.where(lane16 == sub4 * 4 + k, pb16, 0.0), axis=2)
               for k in range(4)]
        sx = 1.0 / (1.0 + jnp.exp(-pbc[0]))
        sy = 1.0 / (1.0 + jnp.exp(-pbc[1]))
        sw = 1.0 / (1.0 + jnp.exp(-pbc[2]))
        sh = 1.0 / (1.0 + jnp.exp(-pbc[3]))
        px = sx * 2.0 - 0.5
        py = sy * 2.0 - 0.5
        pw = (sw * 2.0) ** 2 * aw
        ph = (sh * 2.0) ** 2 * ah
        iou = _ciou(px, py, pw, ph, tbx, tby, gwx, gwy)
        acc[lvl * 6 + _Q_CNT] += jnp.sum(mf)
        acc[lvl * 6 + _Q_BOX] += jnp.sum((1.0 - iou) * mf)
        x = clsg[lvl][...]
        sp = _softplus_part(x)
        cls_iota = lax.broadcasted_iota(i32, (CBLK, 128, CLASS_NUM), 2)
        xl = jnp.sum(jnp.where(cls_iota == label[:, :, None], x, 0.0), axis=2)
        sp_rows = jnp.sum(sp, axis=2)
        acc[lvl * 6 + _Q_SPC] += jnp.sum(sp_rows * mf)
        acc[lvl * 6 + _Q_XLC] += jnp.sum(xl * mf)
        cf16 = cnfg[lvl][...]
        sub16 = (row % 16)[:, :, None]
        xo = jnp.sum(jnp.where(lane16 == sub16, cf16, 0.0), axis=2)
        acc[lvl * 6 + _Q_CORR] += jnp.sum(xo * jnp.clip(iou, 0.0, None) * mf)
        acc[lvl * 6 + _Q_GSP] += jnp.sum(_softplus_part(grids[lvl][...]))

    @pl.when(step == GRID_STEPS - 1)
    def _fin():
        lb = jnp.float32(0.0)
        lo = jnp.float32(0.0)
        lc = jnp.float32(0.0)
        for lvl in range(3):
            cnt = acc[lvl * 6 + _Q_CNT]
            denom = jnp.maximum(cnt, 1.0)
            lb += acc[lvl * 6 + _Q_BOX] / denom
            lc += (acc[lvl * 6 + _Q_SPC] - acc[lvl * 6 + _Q_XLC]) / (
                denom * CLASS_NUM)
            ncell = jnp.float32(NCELLS[lvl])
            lo += ((acc[lvl * 6 + _Q_GSP] - acc[lvl * 6 + _Q_CORR]) / ncell
                   ) * OBJ_SCALE[lvl]
        lb = lb * (BOX_WT * BATCH)
        lo = lo * (OBJ_WT * BATCH)
        lc = lc * (CLS_WT * BATCH)
        li = lax.broadcasted_iota(i32, (8, 128), 1)
        ri = lax.broadcasted_iota(i32, (8, 128), 0)
        out = jnp.where((ri == 0) & (li == 0), lb,
              jnp.where((ri == 0) & (li == 1), lo,
              jnp.where((ri == 0) & (li == 2), lc, 0.0)))
        final_ref[...] = out


def _loss_call(fields, boxg, cnfg, clsg, grids):
    z = np.int32(0)
    cand_spec = pl.BlockSpec((CBLK, 128), lambda i: (i, z))
    wide_spec = pl.BlockSpec((CBLK, 128, 16), lambda i: (i, z, z))
    in_specs = [cand_spec] * 5
    in_specs += [wide_spec for _ in range(3)]
    in_specs += [wide_spec for _ in range(3)]
    in_specs += [pl.BlockSpec((CBLK, 128, CLASS_NUM), lambda i: (i, z, z))
                 for _ in range(3)]
    in_specs += [pl.BlockSpec((GPAD_ROWS[l], 128), lambda i, l=l: (i, z))
                 for l in range(3)]
    return pl.pallas_call(
        _loss_kernel,
        grid=(GRID_STEPS,),
        in_specs=in_specs,
        out_specs=pl.BlockSpec((8, 128), lambda i: (z, z)),
        out_shape=jax.ShapeDtypeStruct((8, 128), f32),
        scratch_shapes=[pltpu.SMEM((32,), f32)],
    )(*fields, *boxg, *cnfg, *clsg, *grids)


def kernel(pred_box_0, pred_cnf_0, pred_cls_0,
           pred_box_1, pred_cnf_1, pred_cls_1,
           pred_box_2, pred_cnf_2, pred_cls_2, targets):
    tg = targets.astype(f32).reshape(NT, 5)
    fields = tuple(jnp.tile(tg[:, k], CAND // NT).reshape(ROWS128, 128)
                   for k in range(5))
    idx9 = _assign_call(fields)
    rows_r = [r.reshape(NW * NCHUNK, CHUNK) for r in idx9[0:3]]
    rows4_r = [r.reshape(NW * NCHUNK, CHUNK) for r in idx9[3:6]]
    rows16_r = [r.reshape(NW * NCHUNK, CHUNK) for r in idx9[6:9]]
    p_box = (pred_box_0, pred_box_1, pred_box_2)
    p_cnf = (pred_cnf_0, pred_cnf_1, pred_cnf_2)
    p_cls = (pred_cls_0, pred_cls_1, pred_cls_2)
    box_fl = [p.reshape(-1, 16) for p in p_box]
    cls_fl = [p.reshape(-1, CLASS_NUM) for p in p_cls]
    cnf_fl = [p.reshape(-1, 16) for p in p_cnf]
    g = _sc_gather_call(rows_r, rows4_r, rows16_r, box_fl, cls_fl, cnf_fl)
    boxg, cnfg, clsg = [], [], []
    for lvl in range(3):
        bg, cg, fg = g[3 * lvl], g[3 * lvl + 1], g[3 * lvl + 2]
        boxg.append(bg.reshape(ROWS128, 128, 16))
        clsg.append(cg.reshape(ROWS128, 128, CLASS_NUM))
        cnfg.append(fg.reshape(ROWS128, 128, 16))
    grids = []
    for lvl in range(3):
        flat = p_cnf[lvl].reshape(-1)
        pad = GPAD_TOT[lvl] * 128 - flat.shape[0]
        if pad:
            flat = jnp.concatenate(
                [flat, jnp.full((pad,), -40.0, dtype=f32)])
        grids.append(flat.reshape(GPAD_TOT[lvl], 128))
    final = _loss_call(fields, boxg, cnfg, clsg, grids)
    return (final[0, 0].reshape(1), final[0, 1].reshape(1),
            final[0, 2].reshape(1))
